# baseline (speedup 1.0000x reference)
import jax
import jax.numpy as jnp
from jax import lax
from jax.experimental import pallas as pl
from jax.experimental.pallas import tpu as pltpu
from jax.experimental.pallas import tpu_sc as plsc

NSEG = 10000
NROW = 160000
D = 256
DH = D // 2
NTEC = 16
CHUNK = 80
ROWS_PER_TEC = NROW // NTEC
NCHUNK = ROWS_PER_TEC // CHUNK
NPASS = 5
PSEG = NSEG // NPASS
PSEG_PAD = 2048
TRASH = PSEG_PAD
ACC_ROWS = PSEG_PAD + 8
SEG_PER_TEC = PSEG_PAD // NTEC
DIVCH = 64
NBUF = 4
LANE = 16


def _seg_mean_body(feats, idxw5, jb_h, z_h, ones_h, out,
                   idx_r, buf, ones_v, jb_v,
                   sem_g, sem_a, sem_c, acc, cnt):
    c = lax.axis_index("c")
    s = lax.axis_index("s")
    row0 = s * ROWS_PER_TEC

    pltpu.sync_copy(jb_h.at[s], jb_v)
    pltpu.sync_copy(ones_h, ones_v)
    jb = jb_v[pl.ds(0, LANE)]

    for p in range(NPASS):
        lo = p * PSEG
        qseg = PSEG

        pltpu.sync_copy(z_h.at[pl.ds(0, SEG_PER_TEC)],
                        acc.at[pl.ds(s * SEG_PER_TEC, SEG_PER_TEC)])
        pltpu.sync_copy(z_h.at[pl.ds(0, SEG_PER_TEC)],
                        cnt.at[pl.ds(s * SEG_PER_TEC, SEG_PER_TEC)])

        @pl.when(s == 0)
        def _trash():
            pltpu.sync_copy(z_h.at[pl.ds(0, 8)], acc.at[pl.ds(TRASH, 8)])
            pltpu.sync_copy(z_h.at[pl.ds(0, 8)], cnt.at[pl.ds(TRASH, 8)])

        plsc.subcore_barrier()

        jlo = jb[2 * p]
        jhi = jb[2 * p + 1]

        def gather(j):
            slot = lax.rem(j, NBUF)
            pltpu.async_copy(
                feats.at[pl.ds(row0 + j * CHUNK, CHUNK), pl.ds(c * DH, DH)],
                buf.at[slot], sem_g.at[slot])
            pltpu.async_copy(idxw5.at[s, p, j], idx_r.at[slot],
                             sem_g.at[slot])

        def drain(j):
            slot = lax.rem(j, NBUF)
            pltpu.make_async_copy(
                buf.at[slot], acc.at[idx_r.at[slot]], sem_a.at[slot]).wait()
            pltpu.make_async_copy(
                ones_v, cnt.at[idx_r.at[slot]], sem_c.at[slot]).wait()

        def chunk_body(j, carry):
            slot = lax.rem(j, NBUF)

            @pl.when(j - 2 >= jlo)
            def _drain_prev():
                drain(j - 2)

            @pl.when(j + 2 < jhi)
            def _next_gather():
                gather(j + 2)

            pltpu.make_async_copy(
                feats.at[pl.ds(row0 + j * CHUNK, CHUNK), pl.ds(c * DH, DH)],
                buf.at[slot], sem_g.at[slot]).wait()
            pltpu.make_async_copy(
                idxw5.at[s, p, j], idx_r.at[slot], sem_g.at[slot]).wait()
            pltpu.async_copy(buf.at[slot], acc.at[idx_r.at[slot]],
                             sem_a.at[slot], add=True)
            pltpu.async_copy(ones_v, cnt.at[idx_r.at[slot]],
                             sem_c.at[slot], add=True)
            return carry

        @pl.when(jlo < jhi)
        def _pro0():
            gather(jlo)

        @pl.when(jlo + 1 < jhi)
        def _pro1():
            gather(jlo + 1)

        lax.fori_loop(jlo, jhi, chunk_body, 0)

        @pl.when(jhi - 2 >= jlo)
        def _epi0():
            drain(jhi - 2)

        @pl.when(jhi - 1 >= jlo)
        def _epi1():
            drain(jhi - 1)

        plsc.subcore_barrier()

        base = s * SEG_PER_TEC
        for q in range(SEG_PER_TEC // DIVCH):
            sub = base + q * DIVCH
            pltpu.sync_copy(acc.at[pl.ds(sub, DIVCH)],
                            buf.at[0, pl.ds(0, DIVCH)])
            pltpu.sync_copy(cnt.at[pl.ds(sub, DIVCH)],
                            buf.at[1, pl.ds(0, DIVCH)])

            def row_body(i, inner):
                scale = 1.0 / jnp.maximum(buf[1, i, pl.ds(0, LANE)], 1.0)
                for k in range(DH // LANE):
                    buf[0, i, pl.ds(k * LANE, LANE)] = (
                        buf[0, i, pl.ds(k * LANE, LANE)] * scale)
                return inner

            lax.fori_loop(0, DIVCH, row_body, 0)
            out_base = lo + sub
            tail_valid = qseg - (PSEG_PAD - DIVCH)
            if q == 0:
                pltpu.sync_copy(
                    buf.at[0, pl.ds(0, DIVCH)],
                    out.at[pl.ds(out_base, DIVCH), pl.ds(c * DH, DH)])
            else:
                @pl.when(s < NTEC - 1)
                def _full():
                    pltpu.sync_copy(
                        buf.at[0, pl.ds(0, DIVCH)],
                        out.at[pl.ds(out_base, DIVCH), pl.ds(c * DH, DH)])

                @pl.when(s == NTEC - 1)
                def _tail():
                    pltpu.sync_copy(
                        buf.at[0, pl.ds(0, tail_valid)],
                        out.at[pl.ds(out_base, tail_valid),
                               pl.ds(c * DH, DH)])

        plsc.subcore_barrier()


def kernel(feats, batch):
    batch32 = batch.astype(jnp.int32)
    batch3d = batch32.reshape(NTEC, NCHUNK, CHUNK)
    firsts = batch3d[:, :, 0]
    lasts = batch3d[:, :, CHUNK - 1]
    cols = []
    remaps = []
    for p in range(NPASS):
        lo = p * PSEG
        hi = lo + PSEG
        cols.append(jnp.sum((lasts < lo).astype(jnp.int32), axis=1))
        cols.append(jnp.sum((firsts < hi).astype(jnp.int32), axis=1))
        in_range = (batch3d >= lo) & (batch3d < hi)
        remaps.append(jnp.where(in_range, batch3d - lo, TRASH))
    jb_h = jnp.stack(
        cols + [jnp.zeros((NTEC,), jnp.int32)] * (LANE - len(cols)),
        axis=1)
    idxw5 = jnp.stack(remaps, axis=1)
    z_h = jnp.zeros((SEG_PER_TEC, DH), jnp.float32)
    ones_h = jnp.ones((CHUNK, DH), jnp.float32)
    f = pl.kernel(
        _seg_mean_body,
        out_type=jax.ShapeDtypeStruct((NSEG, D), jnp.float32),
        mesh=plsc.VectorSubcoreMesh(core_axis_name="c", subcore_axis_name="s"),
        scratch_types=[
            pltpu.VMEM((NBUF, CHUNK), jnp.int32),
            pltpu.VMEM((NBUF, CHUNK, DH), jnp.float32),
            pltpu.VMEM((CHUNK, DH), jnp.float32),
            pltpu.VMEM((LANE,), jnp.int32),
            pltpu.SemaphoreType.DMA((NBUF,)),
            pltpu.SemaphoreType.DMA((NBUF,)),
            pltpu.SemaphoreType.DMA((NBUF,)),
            pltpu.VMEM_SHARED((ACC_ROWS, DH), jnp.float32),
            pltpu.VMEM_SHARED((ACC_ROWS, DH), jnp.float32),
        ],
    )
    return f(feats, idxw5, jb_h, z_h, ones_h)


# speedup vs baseline: 1.0391x; 1.0391x over previous
import jax
import jax.numpy as jnp
from jax import lax
from jax.experimental import pallas as pl
from jax.experimental.pallas import tpu as pltpu
from jax.experimental.pallas import tpu_sc as plsc

NSEG = 10000
NROW = 160000
D = 256
DH = D // 2
NTEC = 16
CHUNK = 80
ROWS_PER_TEC = NROW // NTEC
NCHUNK = ROWS_PER_TEC // CHUNK
NCHUNK_PAD = 128
QBOUNDS = (0, 2504, 5008, 7512, 10000)
QSEG_PAD = 2560
TRASH = QSEG_PAD
ACC_ROWS = QSEG_PAD + 8
SEG_PER_TEC = QSEG_PAD // NTEC
LANE = 16
BIG = 1 << 30


def _seg_mean_body(feats, batch3d, jb_h, z_h, ones_h, out,
                   idx_v, idxw_v, buf, ones_v, jb_v,
                   sem_g, sem_a, sem_c, acc, cnt):
    c = lax.axis_index("c")
    s = lax.axis_index("s")
    row0 = s * ROWS_PER_TEC

    pltpu.sync_copy(batch3d.at[s], idx_v)
    pltpu.sync_copy(jb_h.at[s], jb_v)
    pltpu.sync_copy(ones_h, ones_v)
    jb = jb_v[pl.ds(0, LANE)]

    for p in range(4):
        lo, hi = QBOUNDS[p], QBOUNDS[p + 1]
        qseg = hi - lo

        pltpu.sync_copy(z_h.at[pl.ds(0, SEG_PER_TEC)],
                        acc.at[pl.ds(s * SEG_PER_TEC, SEG_PER_TEC)])
        pltpu.sync_copy(z_h.at[pl.ds(0, SEG_PER_TEC)],
                        cnt.at[pl.ds(s * SEG_PER_TEC, SEG_PER_TEC)])

        @pl.when(s == 0)
        def _trash():
            pltpu.sync_copy(z_h.at[pl.ds(0, 8)], acc.at[pl.ds(TRASH, 8)])
            pltpu.sync_copy(z_h.at[pl.ds(0, 8)], cnt.at[pl.ds(TRASH, 8)])

        plsc.subcore_barrier()

        jlo = jb[2 * p]
        jhi = jb[2 * p + 1]

        def gather(j, par):
            pltpu.async_copy(
                feats.at[pl.ds(row0 + j * CHUNK, CHUNK), pl.ds(c * DH, DH)],
                buf.at[par], sem_g.at[par])

        def chunk_body(j, carry):
            par = lax.rem(j - jlo, 2)
            oth = 1 - par
            for k in range(CHUNK // LANE):
                iv = idx_v[j, pl.ds(k * LANE, LANE)]
                in_range = jnp.logical_and(iv >= lo, iv < hi)
                idxw_v[par, pl.ds(k * LANE, LANE)] = jnp.where(
                    in_range, iv - lo, TRASH)

            @pl.when(j > jlo)
            def _drain_prev():
                pltpu.make_async_copy(
                    buf.at[oth], acc.at[idxw_v.at[oth]], sem_a.at[oth]).wait()
                pltpu.make_async_copy(
                    ones_v, cnt.at[idxw_v.at[oth]], sem_c.at[oth]).wait()

            @pl.when(j + 1 < jhi)
            def _next_gather():
                gather(j + 1, oth)

            pltpu.make_async_copy(
                feats.at[pl.ds(row0 + j * CHUNK, CHUNK), pl.ds(c * DH, DH)],
                buf.at[par], sem_g.at[par]).wait()
            pltpu.async_copy(buf.at[par], acc.at[idxw_v.at[par]],
                             sem_a.at[par], add=True)
            pltpu.async_copy(ones_v, cnt.at[idxw_v.at[par]],
                             sem_c.at[par], add=True)
            return carry

        @pl.when(jhi > jlo)
        def _prologue():
            gather(jlo, 0)

        lax.fori_loop(jlo, jhi, chunk_body, 0)

        @pl.when(jhi > jlo)
        def _epilogue():
            parl = lax.rem(jhi - 1 - jlo, 2)
            pltpu.make_async_copy(
                buf.at[parl], acc.at[idxw_v.at[parl]], sem_a.at[parl]).wait()
            pltpu.make_async_copy(
                ones_v, cnt.at[idxw_v.at[parl]], sem_c.at[parl]).wait()

        plsc.subcore_barrier()

        base = s * SEG_PER_TEC
        for q in range(SEG_PER_TEC // CHUNK):
            sub = base + q * CHUNK
            pltpu.sync_copy(acc.at[pl.ds(sub, CHUNK)], buf.at[0])
            pltpu.sync_copy(cnt.at[pl.ds(sub, CHUNK)], buf.at[1])

            def row_body(i, inner):
                scale = 1.0 / jnp.maximum(buf[1, i, pl.ds(0, LANE)], 1.0)
                for k in range(DH // LANE):
                    buf[0, i, pl.ds(k * LANE, LANE)] = (
                        buf[0, i, pl.ds(k * LANE, LANE)] * scale)
                return inner

            lax.fori_loop(0, CHUNK, row_body, 0)
            out_base = lo + sub
            tail_valid = qseg - (QSEG_PAD - CHUNK)
            if q == 0:
                pltpu.sync_copy(
                    buf.at[0],
                    out.at[pl.ds(out_base, CHUNK), pl.ds(c * DH, DH)])
            else:
                @pl.when(s < NTEC - 1)
                def _full():
                    pltpu.sync_copy(
                        buf.at[0],
                        out.at[pl.ds(out_base, CHUNK), pl.ds(c * DH, DH)])

                @pl.when(s == NTEC - 1)
                def _tail():
                    pltpu.sync_copy(
                        buf.at[0, pl.ds(0, tail_valid)],
                        out.at[pl.ds(out_base, tail_valid),
                               pl.ds(c * DH, DH)])

        plsc.subcore_barrier()


def kernel(feats, batch):
    batch32 = batch.astype(jnp.int32)
    batch3d = batch32.reshape(NTEC, NCHUNK, CHUNK)
    firsts = batch3d[:, :, 0]
    lasts = batch3d[:, :, CHUNK - 1]
    cols = []
    for p in range(4):
        lo, hi = QBOUNDS[p], QBOUNDS[p + 1]
        cols.append(jnp.sum((lasts < lo).astype(jnp.int32), axis=1))
        cols.append(jnp.sum((firsts < hi).astype(jnp.int32), axis=1))
    jb_h = jnp.stack(
        cols + [jnp.zeros((NTEC,), jnp.int32)] * (LANE - len(cols)),
        axis=1)
    z_h = jnp.zeros((SEG_PER_TEC, DH), jnp.float32)
    ones_h = jnp.ones((CHUNK, DH), jnp.float32)
    f = pl.kernel(
        _seg_mean_body,
        out_type=jax.ShapeDtypeStruct((NSEG, D), jnp.float32),
        mesh=plsc.VectorSubcoreMesh(core_axis_name="c", subcore_axis_name="s"),
        scratch_types=[
            pltpu.VMEM((NCHUNK, CHUNK), jnp.int32),
            pltpu.VMEM((2, CHUNK), jnp.int32),
            pltpu.VMEM((2, CHUNK, DH), jnp.float32),
            pltpu.VMEM((CHUNK, DH), jnp.float32),
            pltpu.VMEM((LANE,), jnp.int32),
            pltpu.SemaphoreType.DMA((2,)),
            pltpu.SemaphoreType.DMA((2,)),
            pltpu.SemaphoreType.DMA((2,)),
            pltpu.VMEM_SHARED((ACC_ROWS, DH), jnp.float32),
            pltpu.VMEM_SHARED((ACC_ROWS, DH), jnp.float32),
        ],
    )
    return f(feats, batch3d, jb_h, z_h, ones_h)
